# baseline (speedup 1.0000x reference)
import dataclasses
import functools

import jax
import jax.numpy as jnp
from jax import lax
from jax.experimental import pallas as pl
from jax.experimental.pallas import tpu as pltpu
from jax.experimental.pallas import tpu_sc as plsc

_N = 4096
_M = 8
_TOPK = 10
_L = 16
_NW = 16
_SLICE = _N // _NW
_KV = _SLICE // _L
_NEG = -1.0
_BIGI = 1 << 30
_TAIL = 3 * _TOPK + _M
_OUT = 3 * _N + _TAIL


def _sc_body(x_hbm, prev_hbm, m_hbm, out_hbm,
             xv, pv, mv, stage, nr, tvr, tir, vb, ib, tail,
             sh_vals, sh_idx, sh_niche, sem):
    cid = lax.axis_index("c")
    sid = lax.axis_index("s")
    lanes = lax.iota(jnp.int32, _L)
    lanes8 = lanes * _M

    @pl.when(cid == 0)
    def _():
        w = sid
        base = w * _SLICE
        pltpu.async_copy(x_hbm.at[pl.ds(base, _SLICE)], xv, sem).wait()
        pltpu.async_copy(prev_hbm.at[pl.ds(base, _SLICE)], pv, sem).wait()
        pltpu.async_copy(m_hbm.at[pl.ds(base * _M, _SLICE * _M)], mv,
                         sem).wait()

        cks = []
        tvals = jnp.full((_L,), _NEG, jnp.float32)
        tidx = jnp.zeros((_L,), jnp.int32)
        for k in range(_KV):
            ck = xv[pl.ds(k * _L, _L)] - pv[pl.ds(k * _L, _L)]
            cks.append(ck)
            sv, si = plsc.sort_key_val(jnp.abs(ck), lanes + (base + k * _L),
                                       descending=True)
            take = sv > tvals
            nv = jnp.where(take, sv, tvals)
            ni = jnp.where(take, si, tidx)
            tvals, tidx = plsc.sort_key_val(nv, ni)
            stage[pl.ds(k * _L, _L)] = ck

        pltpu.async_copy(stage, out_hbm.at[pl.ds(base, _SLICE)], sem).wait()
        tvr[...] = tvals
        tir[...] = tidx

        part = jnp.zeros((_L,), jnp.float32)
        for j in range(_M):
            acc = jnp.zeros((_L,), jnp.float32)
            for k in range(_KV):
                g = plsc.load_gather(mv, [lanes8 + (k * _L * _M + j)])
                acc = acc + cks[k] * g
            part = jnp.where(lanes == j, jnp.sum(acc), part)
        nr[...] = part

        pltpu.sync_copy(nr, sh_niche.at[pl.ds(w * _L, _L)])
        pltpu.sync_copy(tvr, sh_vals.at[pl.ds(w * _L, _L)])
        pltpu.sync_copy(tir, sh_idx.at[pl.ds(w * _L, _L)])
        plsc.subcore_barrier()

        pltpu.sync_copy(sh_niche, vb)
        niche = vb[pl.ds(0, _L)]
        for b in range(1, _NW):
            niche = niche + vb[pl.ds(b * _L, _L)]
        nr[...] = niche

        njs = [plsc.load_gather(nr, [jnp.full((_L,), j, jnp.int32)])
               for j in range(_M)]
        for k in range(_KV):
            ek = jnp.zeros((_L,), jnp.float32)
            for j in range(_M):
                g = plsc.load_gather(mv, [lanes8 + (k * _L * _M + j)])
                ek = ek + njs[j] * g
            stage[pl.ds(k * _L, _L)] = ek
            xv[pl.ds(k * _L, _L)] = cks[k] - ek
        pltpu.sync_copy(stage, out_hbm.at[pl.ds(_N + base, _SLICE)])
        pltpu.sync_copy(xv, out_hbm.at[pl.ds(2 * _N + base, _SLICE)])

        @pl.when(w == 0)
        def _():
            pltpu.sync_copy(sh_vals, vb)
            pltpu.sync_copy(sh_idx, ib)
            tv = jnp.full((_L,), _NEG, jnp.float32)
            ti = jnp.zeros((_L,), jnp.int32)
            for b in range(_NW):
                sv, si = plsc.sort_key_val(vb[pl.ds(b * _L, _L)],
                                           ib[pl.ds(b * _L, _L)],
                                           descending=True)
                take = sv > tv
                nv = jnp.where(take, sv, tv)
                ni = jnp.where(take, si, ti)
                tv, ti = plsc.sort_key_val(nv, ni)
            tvr[...] = tv
            tir[...] = ti

            prods = []
            keys = []
            keybase = ti * _N
            for s in range(1, _L):
                perm = (lanes + s) & (_L - 1)
                rv = plsc.load_gather(tvr, [perm])
                ri = plsc.load_gather(tir, [perm])
                prods.append(tv * rv)
                keys.append(keybase + ri)

            selv = jnp.zeros((_L,), jnp.float32)
            selk = jnp.zeros((_L,), jnp.int32)
            first_m = None
            for t in range(_TOPK):
                m = prods[0]
                for s in range(1, _L - 1):
                    m = jnp.maximum(m, prods[s])
                ms = jnp.max(m)
                if first_m is None:
                    first_m = ms
                msv = jnp.full((_L,), ms)
                kc = jnp.full((_L,), _BIGI, jnp.int32)
                for s in range(_L - 1):
                    kc = jnp.minimum(
                        kc, jnp.where(prods[s] == msv, keys[s], _BIGI))
                km = jnp.min(kc)
                kmv = jnp.full((_L,), km)
                for s in range(_L - 1):
                    prods[s] = jnp.where(keys[s] == kmv, _NEG, prods[s])
                selv = jnp.where(lanes == t, msv, selv)
                selk = jnp.where(lanes == t, kmv, selk)

            inv_max = jnp.full((_L,), jnp.maximum(first_m, 1e-8))
            tmask = lanes < _TOPK
            nmask = lanes < _M
            ti3 = jnp.where(tmask, lanes * 3, 0)
            tin = jnp.where(nmask, lanes + 3 * _TOPK, 0)
            plsc.store_scatter(tail, [ti3 + 2], selv / inv_max, mask=tmask)
            plsc.store_scatter(tail, [ti3],
                               (selk >> 12).astype(jnp.float32), mask=tmask)
            plsc.store_scatter(tail, [ti3 + 1],
                               (selk & (_N - 1)).astype(jnp.float32),
                               mask=tmask)
            plsc.store_scatter(tail, [tin], nr[...], mask=nmask)
            pltpu.sync_copy(tail, out_hbm.at[pl.ds(3 * _N, _TAIL)])


@functools.cache
def _sc_kernel():
    mesh = plsc.VectorSubcoreMesh(core_axis_name="c", subcore_axis_name="s")
    cp = pltpu.CompilerParams()
    if "needs_layout_passes" in pltpu.CompilerParams.__dataclass_fields__:
        cp = dataclasses.replace(cp, needs_layout_passes=False)
    return pl.kernel(
        _sc_body,
        mesh=mesh,
        compiler_params=cp,
        out_type=jax.ShapeDtypeStruct((_OUT,), jnp.float32),
        scratch_types=[
            pltpu.VMEM((_SLICE,), jnp.float32),
            pltpu.VMEM((_SLICE,), jnp.float32),
            pltpu.VMEM((_SLICE * _M,), jnp.float32),
            pltpu.VMEM((_SLICE,), jnp.float32),
            pltpu.VMEM((_L,), jnp.float32),
            pltpu.VMEM((_L,), jnp.float32),
            pltpu.VMEM((_L,), jnp.int32),
            pltpu.VMEM((_NW * _L,), jnp.float32),
            pltpu.VMEM((_NW * _L,), jnp.int32),
            pltpu.VMEM((_TAIL,), jnp.float32),
            pltpu.VMEM_SHARED((_NW * _L,), jnp.float32),
            pltpu.VMEM_SHARED((_NW * _L,), jnp.int32),
            pltpu.VMEM_SHARED((_NW * _L,), jnp.float32),
            pltpu.SemaphoreType.DMA,
        ],
    )


@jax.jit
def kernel(current_neuron_state, mechanism_state, prev_state):
    return _sc_kernel()(current_neuron_state, prev_state,
                        mechanism_state.reshape(-1))


# speedup vs baseline: 1.0508x; 1.0508x over previous
import dataclasses
import functools

import jax
import jax.numpy as jnp
from jax import lax
from jax.experimental import pallas as pl
from jax.experimental.pallas import tpu as pltpu
from jax.experimental.pallas import tpu_sc as plsc

_N = 4096
_M = 8
_TOPK = 10
_L = 16
_NW = 16
_SLICE = _N // _NW
_KV = _SLICE // _L
_NEG = -1.0
_BIGI = 1 << 30
_TAIL = 3 * _TOPK + _M
_OUT = 3 * _N + _TAIL


def _sc_body(x_hbm, prev_hbm, m_hbm, out_hbm,
             xv, pv, mv, stage, nr, tvr, tir, vb, ib, tail,
             sh_vals, sh_idx, sh_niche, sem, semm, semo):
    cid = lax.axis_index("c")
    sid = lax.axis_index("s")
    lanes = lax.iota(jnp.int32, _L)
    lanes8 = lanes * _M

    @pl.when(cid == 0)
    def _():
        w = sid
        base = w * _SLICE
        cm = pltpu.async_copy(m_hbm.at[pl.ds(base * _M, _SLICE * _M)], mv,
                              semm)
        cx = pltpu.async_copy(x_hbm.at[pl.ds(base, _SLICE)], xv, sem)
        cp_ = pltpu.async_copy(prev_hbm.at[pl.ds(base, _SLICE)], pv, sem)
        cx.wait()
        cp_.wait()

        cks = []
        tvals = jnp.full((_L,), _NEG, jnp.float32)
        tidx = jnp.zeros((_L,), jnp.int32)
        for k in range(_KV):
            ck = xv[pl.ds(k * _L, _L)] - pv[pl.ds(k * _L, _L)]
            cks.append(ck)
            sv, si = plsc.sort_key_val(jnp.abs(ck), lanes + (base + k * _L),
                                       descending=True)
            take = sv > tvals
            nv = jnp.where(take, sv, tvals)
            ni = jnp.where(take, si, tidx)
            tvals, tidx = plsc.sort_key_val(nv, ni)
            stage[pl.ds(k * _L, _L)] = ck

        co = pltpu.async_copy(stage, out_hbm.at[pl.ds(base, _SLICE)], semo)
        tvr[...] = tvals
        tir[...] = tidx
        cm.wait()

        part = jnp.zeros((_L,), jnp.float32)
        for j in range(_M):
            acc = jnp.zeros((_L,), jnp.float32)
            for k in range(_KV):
                g = plsc.load_gather(mv, [lanes8 + (k * _L * _M + j)])
                acc = acc + cks[k] * g
            part = jnp.where(lanes == j, jnp.sum(acc), part)
        nr[...] = part
        co.wait()

        pltpu.sync_copy(nr, sh_niche.at[pl.ds(w * _L, _L)])
        pltpu.sync_copy(tvr, sh_vals.at[pl.ds(w * _L, _L)])
        pltpu.sync_copy(tir, sh_idx.at[pl.ds(w * _L, _L)])
        plsc.subcore_barrier()

        pltpu.sync_copy(sh_niche, vb)
        niche = vb[pl.ds(0, _L)]
        for b in range(1, _NW):
            niche = niche + vb[pl.ds(b * _L, _L)]
        nr[...] = niche

        njs = [plsc.load_gather(nr, [jnp.full((_L,), j, jnp.int32)])
               for j in range(_M)]
        for k in range(_KV):
            ek = jnp.zeros((_L,), jnp.float32)
            for j in range(_M):
                g = plsc.load_gather(mv, [lanes8 + (k * _L * _M + j)])
                ek = ek + njs[j] * g
            stage[pl.ds(k * _L, _L)] = ek
            xv[pl.ds(k * _L, _L)] = cks[k] - ek
        pltpu.sync_copy(stage, out_hbm.at[pl.ds(_N + base, _SLICE)])
        pltpu.sync_copy(xv, out_hbm.at[pl.ds(2 * _N + base, _SLICE)])

        @pl.when(w == 0)
        def _():
            pltpu.sync_copy(sh_vals, vb)
            pltpu.sync_copy(sh_idx, ib)
            tv = jnp.full((_L,), _NEG, jnp.float32)
            ti = jnp.zeros((_L,), jnp.int32)
            for b in range(_NW):
                sv, si = plsc.sort_key_val(vb[pl.ds(b * _L, _L)],
                                           ib[pl.ds(b * _L, _L)],
                                           descending=True)
                take = sv > tv
                nv = jnp.where(take, sv, tv)
                ni = jnp.where(take, si, ti)
                tv, ti = plsc.sort_key_val(nv, ni)
            tvr[...] = tv
            tir[...] = ti

            prods = []
            keys = []
            keybase = ti * _N
            for s in range(1, _L):
                perm = (lanes + s) & (_L - 1)
                rv = plsc.load_gather(tvr, [perm])
                ri = plsc.load_gather(tir, [perm])
                prods.append(tv * rv)
                keys.append(keybase + ri)

            selv = jnp.zeros((_L,), jnp.float32)
            selk = jnp.zeros((_L,), jnp.int32)
            first_m = None
            for t in range(_TOPK):
                m = prods[0]
                for s in range(1, _L - 1):
                    m = jnp.maximum(m, prods[s])
                ms = jnp.max(m)
                if first_m is None:
                    first_m = ms
                msv = jnp.full((_L,), ms)
                kc = jnp.full((_L,), _BIGI, jnp.int32)
                for s in range(_L - 1):
                    kc = jnp.minimum(
                        kc, jnp.where(prods[s] == msv, keys[s], _BIGI))
                km = jnp.min(kc)
                kmv = jnp.full((_L,), km)
                for s in range(_L - 1):
                    prods[s] = jnp.where(keys[s] == kmv, _NEG, prods[s])
                selv = jnp.where(lanes == t, msv, selv)
                selk = jnp.where(lanes == t, kmv, selk)

            inv_max = jnp.full((_L,), jnp.maximum(first_m, 1e-8))
            tmask = lanes < _TOPK
            nmask = lanes < _M
            ti3 = jnp.where(tmask, lanes * 3, 0)
            tin = jnp.where(nmask, lanes + 3 * _TOPK, 0)
            plsc.store_scatter(tail, [ti3 + 2], selv / inv_max, mask=tmask)
            plsc.store_scatter(tail, [ti3],
                               (selk >> 12).astype(jnp.float32), mask=tmask)
            plsc.store_scatter(tail, [ti3 + 1],
                               (selk & (_N - 1)).astype(jnp.float32),
                               mask=tmask)
            plsc.store_scatter(tail, [tin], nr[...], mask=nmask)
            pltpu.sync_copy(tail, out_hbm.at[pl.ds(3 * _N, _TAIL)])


@functools.cache
def _sc_kernel():
    mesh = plsc.VectorSubcoreMesh(core_axis_name="c", subcore_axis_name="s")
    cp = pltpu.CompilerParams()
    if "needs_layout_passes" in pltpu.CompilerParams.__dataclass_fields__:
        cp = dataclasses.replace(cp, needs_layout_passes=False)
    return pl.kernel(
        _sc_body,
        mesh=mesh,
        compiler_params=cp,
        out_type=jax.ShapeDtypeStruct((_OUT,), jnp.float32),
        scratch_types=[
            pltpu.VMEM((_SLICE,), jnp.float32),
            pltpu.VMEM((_SLICE,), jnp.float32),
            pltpu.VMEM((_SLICE * _M,), jnp.float32),
            pltpu.VMEM((_SLICE,), jnp.float32),
            pltpu.VMEM((_L,), jnp.float32),
            pltpu.VMEM((_L,), jnp.float32),
            pltpu.VMEM((_L,), jnp.int32),
            pltpu.VMEM((_NW * _L,), jnp.float32),
            pltpu.VMEM((_NW * _L,), jnp.int32),
            pltpu.VMEM((_TAIL,), jnp.float32),
            pltpu.VMEM_SHARED((_NW * _L,), jnp.float32),
            pltpu.VMEM_SHARED((_NW * _L,), jnp.int32),
            pltpu.VMEM_SHARED((_NW * _L,), jnp.float32),
            pltpu.SemaphoreType.DMA,
            pltpu.SemaphoreType.DMA,
            pltpu.SemaphoreType.DMA,
        ],
    )


@jax.jit
def kernel(current_neuron_state, mechanism_state, prev_state):
    return _sc_kernel()(current_neuron_state, prev_state,
                        mechanism_state.reshape(-1))


# speedup vs baseline: 1.1284x; 1.0738x over previous
import dataclasses
import functools

import jax
import jax.numpy as jnp
from jax import lax
from jax.experimental import pallas as pl
from jax.experimental.pallas import tpu as pltpu
from jax.experimental.pallas import tpu_sc as plsc

_N = 4096
_M = 8
_TOPK = 10
_L = 16
_NW = 16
_SLICE = _N // _NW
_KV = _SLICE // _L
_NEG = -1.0
_BIGI = 1 << 30
_NPAIR = _L - 1
_TAIL = 3 * _TOPK + _M
_OUT = 3 * _N + _TAIL


def _sc_body(x_hbm, prev_hbm, m_hbm, out_hbm,
             xv, pv, mv, stage, nr, tvr, tir, vb, ib, tail, pr, kr,
             sh_vals, sh_idx, sh_niche, sem, semm, semo):
    cid = lax.axis_index("c")
    sid = lax.axis_index("s")
    lanes = lax.iota(jnp.int32, _L)
    lanes8 = lanes * _M

    @pl.when(cid == 0)
    def _():
        w = sid
        base = w * _SLICE
        cm = pltpu.async_copy(m_hbm.at[pl.ds(base * _M, _SLICE * _M)], mv,
                              semm)
        cx = pltpu.async_copy(x_hbm.at[pl.ds(base, _SLICE)], xv, sem)
        cp_ = pltpu.async_copy(prev_hbm.at[pl.ds(base, _SLICE)], pv, sem)
        cx.wait()
        cp_.wait()

        def scan_step(k, carry):
            tvals, tidx = carry
            off = k * _L
            ck = xv[pl.ds(off, _L)] - pv[pl.ds(off, _L)]
            stage[pl.ds(off, _L)] = ck
            sv, si = plsc.sort_key_val(jnp.abs(ck), lanes + (base + off),
                                       descending=True)
            take = sv > tvals
            nv = jnp.where(take, sv, tvals)
            ni = jnp.where(take, si, tidx)
            nv2, ni2 = plsc.sort_key_val(nv, ni)
            return (nv2, ni2)

        tvals, tidx = lax.fori_loop(
            0, _KV, scan_step,
            (jnp.full((_L,), _NEG, jnp.float32), jnp.zeros((_L,), jnp.int32)))

        co = pltpu.async_copy(stage, out_hbm.at[pl.ds(base, _SLICE)], semo)
        tvr[...] = tvals
        tir[...] = tidx
        cm.wait()

        def niche_step(k, accs):
            off = k * _L
            ck = stage[pl.ds(off, _L)]
            moff = lanes8 + off * _M
            return tuple(
                accs[j] + ck * plsc.load_gather(mv, [moff + j])
                for j in range(_M))

        accs = lax.fori_loop(
            0, _KV, niche_step,
            tuple(jnp.zeros((_L,), jnp.float32) for _ in range(_M)))
        part = jnp.zeros((_L,), jnp.float32)
        for j in range(_M):
            part = jnp.where(lanes == j, jnp.sum(accs[j]), part)
        nr[...] = part
        co.wait()

        pltpu.sync_copy(nr, sh_niche.at[pl.ds(w * _L, _L)])
        pltpu.sync_copy(tvr, sh_vals.at[pl.ds(w * _L, _L)])
        pltpu.sync_copy(tir, sh_idx.at[pl.ds(w * _L, _L)])
        plsc.subcore_barrier()

        pltpu.sync_copy(sh_niche, vb)

        def nred_step(b, acc):
            return acc + vb[pl.ds(b * _L, _L)]

        niche = lax.fori_loop(0, _NW, nred_step,
                              jnp.zeros((_L,), jnp.float32))
        nr[...] = niche

        njs = [plsc.load_gather(nr, [jnp.full((_L,), j, jnp.int32)])
               for j in range(_M)]

        @pl.loop(0, _KV)
        def _(k):
            off = k * _L
            ck = stage[pl.ds(off, _L)]
            moff = lanes8 + off * _M
            ek = njs[0] * plsc.load_gather(mv, [moff])
            for j in range(1, _M):
                ek = ek + njs[j] * plsc.load_gather(mv, [moff + j])
            stage[pl.ds(off, _L)] = ek
            xv[pl.ds(off, _L)] = ck - ek

        pltpu.sync_copy(stage, out_hbm.at[pl.ds(_N + base, _SLICE)])
        pltpu.sync_copy(xv, out_hbm.at[pl.ds(2 * _N + base, _SLICE)])

        @pl.when(w == 0)
        def _():
            pltpu.sync_copy(sh_vals, vb)
            pltpu.sync_copy(sh_idx, ib)

            def merge_step(b, carry):
                tv, ti = carry
                sv, si = plsc.sort_key_val(vb[pl.ds(b * _L, _L)],
                                           ib[pl.ds(b * _L, _L)],
                                           descending=True)
                take = sv > tv
                nv = jnp.where(take, sv, tv)
                ni = jnp.where(take, si, ti)
                nv2, ni2 = plsc.sort_key_val(nv, ni)
                return (nv2, ni2)

            tv, ti = lax.fori_loop(
                0, _NW, merge_step,
                (jnp.full((_L,), _NEG, jnp.float32),
                 jnp.zeros((_L,), jnp.int32)))
            tvr[...] = tv
            tir[...] = ti

            keybase = ti * _N

            @pl.loop(1, _L)
            def _(s):
                perm = (lanes + s) & (_L - 1)
                rv = plsc.load_gather(tvr, [perm])
                ri = plsc.load_gather(tir, [perm])
                off = (s - 1) * _L
                pr[pl.ds(off, _L)] = tv * rv
                kr[pl.ds(off, _L)] = keybase + ri

            def top_step(t, carry):
                selv, selk, maxv = carry

                def max_step(s, mx):
                    return jnp.maximum(mx, pr[pl.ds(s * _L, _L)])

                m = lax.fori_loop(0, _NPAIR, max_step,
                                  jnp.full((_L,), _NEG, jnp.float32))
                ms = jnp.max(m)
                msv = jnp.full((_L,), ms)

                def key_step(s, kc):
                    off = s * _L
                    return jnp.minimum(
                        kc, jnp.where(pr[pl.ds(off, _L)] == msv,
                                      kr[pl.ds(off, _L)], _BIGI))

                kc = lax.fori_loop(0, _NPAIR, key_step,
                                   jnp.full((_L,), _BIGI, jnp.int32))
                km = jnp.min(kc)
                kmv = jnp.full((_L,), km)

                @pl.loop(0, _NPAIR)
                def _(s):
                    off = s * _L
                    pr[pl.ds(off, _L)] = jnp.where(
                        kr[pl.ds(off, _L)] == kmv, _NEG, pr[pl.ds(off, _L)])

                selv = jnp.where(lanes == t, msv, selv)
                selk = jnp.where(lanes == t, kmv, selk)
                maxv = jnp.where(t == 0, ms, maxv)
                return (selv, selk, maxv)

            selv, selk, maxv = lax.fori_loop(
                0, _TOPK, top_step,
                (jnp.zeros((_L,), jnp.float32), jnp.zeros((_L,), jnp.int32),
                 jnp.float32(0.0)))

            inv_max = jnp.full((_L,), jnp.maximum(maxv, 1e-8))
            tmask = lanes < _TOPK
            nmask = lanes < _M
            ti3 = jnp.where(tmask, lanes * 3, 0)
            tin = jnp.where(nmask, lanes + 3 * _TOPK, 0)
            plsc.store_scatter(tail, [ti3 + 2], selv / inv_max, mask=tmask)
            plsc.store_scatter(tail, [ti3],
                               (selk >> 12).astype(jnp.float32), mask=tmask)
            plsc.store_scatter(tail, [ti3 + 1],
                               (selk & (_N - 1)).astype(jnp.float32),
                               mask=tmask)
            plsc.store_scatter(tail, [tin], nr[...], mask=nmask)
            pltpu.sync_copy(tail, out_hbm.at[pl.ds(3 * _N, _TAIL)])


@functools.cache
def _sc_kernel():
    mesh = plsc.VectorSubcoreMesh(core_axis_name="c", subcore_axis_name="s")
    cp = pltpu.CompilerParams()
    if "needs_layout_passes" in pltpu.CompilerParams.__dataclass_fields__:
        cp = dataclasses.replace(cp, needs_layout_passes=False)
    return pl.kernel(
        _sc_body,
        mesh=mesh,
        compiler_params=cp,
        out_type=jax.ShapeDtypeStruct((_OUT,), jnp.float32),
        scratch_types=[
            pltpu.VMEM((_SLICE,), jnp.float32),
            pltpu.VMEM((_SLICE,), jnp.float32),
            pltpu.VMEM((_SLICE * _M,), jnp.float32),
            pltpu.VMEM((_SLICE,), jnp.float32),
            pltpu.VMEM((_L,), jnp.float32),
            pltpu.VMEM((_L,), jnp.float32),
            pltpu.VMEM((_L,), jnp.int32),
            pltpu.VMEM((_NW * _L,), jnp.float32),
            pltpu.VMEM((_NW * _L,), jnp.int32),
            pltpu.VMEM((_TAIL,), jnp.float32),
            pltpu.VMEM((_NPAIR * _L,), jnp.float32),
            pltpu.VMEM((_NPAIR * _L,), jnp.int32),
            pltpu.VMEM_SHARED((_NW * _L,), jnp.float32),
            pltpu.VMEM_SHARED((_NW * _L,), jnp.int32),
            pltpu.VMEM_SHARED((_NW * _L,), jnp.float32),
            pltpu.SemaphoreType.DMA,
            pltpu.SemaphoreType.DMA,
            pltpu.SemaphoreType.DMA,
        ],
    )


@jax.jit
def kernel(current_neuron_state, mechanism_state, prev_state):
    return _sc_kernel()(current_neuron_state, prev_state,
                        mechanism_state.reshape(-1))


# speedup vs baseline: 1.1549x; 1.0234x over previous
import dataclasses
import functools

import jax
import jax.numpy as jnp
from jax import lax
from jax.experimental import pallas as pl
from jax.experimental.pallas import tpu as pltpu
from jax.experimental.pallas import tpu_sc as plsc

_N = 4096
_M = 8
_TOPK = 10
_L = 16
_NW = 16
_SLICE = _N // _NW
_KV = _SLICE // _L
_NEG = -1.0
_BIGI = 1 << 30
_NH = 8
_NPAIR = _NH - 1
_TAIL = 3 * _TOPK + _M
_OUT = 3 * _N + _TAIL


def _sc_body(x_hbm, prev_hbm, m_hbm, out_hbm,
             xv, pv, mv, stage, nr, tvr, tir, vb, ib, tail, pr, kr,
             sh_vals, sh_idx, sh_niche, sem, semm, semo):
    cid = lax.axis_index("c")
    sid = lax.axis_index("s")
    lanes = lax.iota(jnp.int32, _L)
    lanes8 = lanes * _M

    @pl.when(cid == 0)
    def _():
        w = sid
        base = w * _SLICE
        cm = pltpu.async_copy(m_hbm.at[pl.ds(base * _M, _SLICE * _M)], mv,
                              semm)
        cx = pltpu.async_copy(x_hbm.at[pl.ds(base, _SLICE)], xv, sem)
        cp_ = pltpu.async_copy(prev_hbm.at[pl.ds(base, _SLICE)], pv, sem)
        cx.wait()
        cp_.wait()

        def scan_step(k, carry):
            tvals, tidx = carry
            off = k * _L
            ck = xv[pl.ds(off, _L)] - pv[pl.ds(off, _L)]
            stage[pl.ds(off, _L)] = ck
            sv, si = plsc.sort_key_val(jnp.abs(ck), lanes + (base + off),
                                       descending=True)
            take = sv > tvals
            nv = jnp.where(take, sv, tvals)
            ni = jnp.where(take, si, tidx)
            nv2, ni2 = plsc.sort_key_val(nv, ni)
            return (nv2, ni2)

        tvals, tidx = lax.fori_loop(
            0, _KV, scan_step,
            (jnp.full((_L,), _NEG, jnp.float32), jnp.zeros((_L,), jnp.int32)))

        co = pltpu.async_copy(stage, out_hbm.at[pl.ds(base, _SLICE)], semo)
        tvr[...] = tvals
        tir[...] = tidx
        cm.wait()

        def niche_step(k, accs):
            off = k * _L
            ck = stage[pl.ds(off, _L)]
            moff = lanes8 + off * _M
            return tuple(
                accs[j] + ck * plsc.load_gather(mv, [moff + j])
                for j in range(_M))

        accs = lax.fori_loop(
            0, _KV, niche_step,
            tuple(jnp.zeros((_L,), jnp.float32) for _ in range(_M)))
        part = jnp.zeros((_L,), jnp.float32)
        for j in range(_M):
            part = jnp.where(lanes == j, jnp.sum(accs[j]), part)
        nr[...] = part
        co.wait()

        pltpu.sync_copy(nr, sh_niche.at[pl.ds(w * _L, _L)])
        pltpu.sync_copy(tvr, sh_vals.at[pl.ds(w * _L, _L)])
        pltpu.sync_copy(tir, sh_idx.at[pl.ds(w * _L, _L)])
        plsc.subcore_barrier()

        pltpu.sync_copy(sh_niche, vb)

        def nred_step(b, acc):
            return acc + vb[pl.ds(b * _L, _L)]

        niche = lax.fori_loop(0, _NW, nred_step,
                              jnp.zeros((_L,), jnp.float32))
        nr[...] = niche

        njs = [plsc.load_gather(nr, [jnp.full((_L,), j, jnp.int32)])
               for j in range(_M)]

        @pl.loop(0, _KV)
        def _(k):
            off = k * _L
            ck = stage[pl.ds(off, _L)]
            moff = lanes8 + off * _M
            ek = njs[0] * plsc.load_gather(mv, [moff])
            for j in range(1, _M):
                ek = ek + njs[j] * plsc.load_gather(mv, [moff + j])
            stage[pl.ds(off, _L)] = ek
            xv[pl.ds(off, _L)] = ck - ek

        ce = pltpu.async_copy(stage, out_hbm.at[pl.ds(_N + base, _SLICE)],
                              semo)
        cr = pltpu.async_copy(xv, out_hbm.at[pl.ds(2 * _N + base, _SLICE)],
                              semo)

        @pl.when(w == 0)
        def _():
            pltpu.sync_copy(sh_vals, vb)
            pltpu.sync_copy(sh_idx, ib)

            def merge_step(b, carry):
                tv, ti = carry
                sv, si = plsc.sort_key_val(vb[pl.ds(b * _L, _L)],
                                           ib[pl.ds(b * _L, _L)],
                                           descending=True)
                take = sv > tv
                nv = jnp.where(take, sv, tv)
                ni = jnp.where(take, si, ti)
                nv2, ni2 = plsc.sort_key_val(nv, ni)
                return (nv2, ni2)

            tv, ti = lax.fori_loop(
                0, _NW, merge_step,
                (jnp.full((_L,), _NEG, jnp.float32),
                 jnp.zeros((_L,), jnp.int32)))
            tvr[...] = tv
            tir[...] = ti

            keybase = ti * _N
            hi = lanes >= _NH

            @pl.loop(1, _NH)
            def _(s):
                perm = ((lanes + s) & (_NH - 1)) | _NH
                rv = plsc.load_gather(tvr, [perm])
                ri = plsc.load_gather(tir, [perm])
                off = (s - 1) * _L
                pr[pl.ds(off, _L)] = jnp.where(hi, tv * rv, _NEG)
                kr[pl.ds(off, _L)] = keybase + ri

            def top_step(t, carry):
                selv, selk, maxv = carry

                def max_step(s, mx):
                    return jnp.maximum(mx, pr[pl.ds(s * _L, _L)])

                m = lax.fori_loop(0, _NPAIR, max_step,
                                  jnp.full((_L,), _NEG, jnp.float32))
                ms = jnp.max(m)
                msv = jnp.full((_L,), ms)

                def key_step(s, kc):
                    off = s * _L
                    return jnp.minimum(
                        kc, jnp.where(pr[pl.ds(off, _L)] == msv,
                                      kr[pl.ds(off, _L)], _BIGI))

                kc = lax.fori_loop(0, _NPAIR, key_step,
                                   jnp.full((_L,), _BIGI, jnp.int32))
                km = jnp.min(kc)
                kmv = jnp.full((_L,), km)

                @pl.loop(0, _NPAIR)
                def _(s):
                    off = s * _L
                    pr[pl.ds(off, _L)] = jnp.where(
                        kr[pl.ds(off, _L)] == kmv, _NEG, pr[pl.ds(off, _L)])

                selv = jnp.where(lanes == t, msv, selv)
                selk = jnp.where(lanes == t, kmv, selk)
                maxv = jnp.where(t == 0, ms, maxv)
                return (selv, selk, maxv)

            selv, selk, maxv = lax.fori_loop(
                0, _TOPK, top_step,
                (jnp.zeros((_L,), jnp.float32), jnp.zeros((_L,), jnp.int32),
                 jnp.float32(0.0)))

            inv_max = jnp.full((_L,), jnp.maximum(maxv, 1e-8))
            tmask = lanes < _TOPK
            nmask = lanes < _M
            ti3 = jnp.where(tmask, lanes * 3, 0)
            tin = jnp.where(nmask, lanes + 3 * _TOPK, 0)
            plsc.store_scatter(tail, [ti3 + 2], selv / inv_max, mask=tmask)
            plsc.store_scatter(tail, [ti3],
                               (selk >> 12).astype(jnp.float32), mask=tmask)
            plsc.store_scatter(tail, [ti3 + 1],
                               (selk & (_N - 1)).astype(jnp.float32),
                               mask=tmask)
            plsc.store_scatter(tail, [tin], nr[...], mask=nmask)
            pltpu.sync_copy(tail, out_hbm.at[pl.ds(3 * _N, _TAIL)])

        ce.wait()
        cr.wait()


@functools.cache
def _sc_kernel():
    mesh = plsc.VectorSubcoreMesh(core_axis_name="c", subcore_axis_name="s")
    cp = pltpu.CompilerParams()
    if "needs_layout_passes" in pltpu.CompilerParams.__dataclass_fields__:
        cp = dataclasses.replace(cp, needs_layout_passes=False)
    return pl.kernel(
        _sc_body,
        mesh=mesh,
        compiler_params=cp,
        out_type=jax.ShapeDtypeStruct((_OUT,), jnp.float32),
        scratch_types=[
            pltpu.VMEM((_SLICE,), jnp.float32),
            pltpu.VMEM((_SLICE,), jnp.float32),
            pltpu.VMEM((_SLICE * _M,), jnp.float32),
            pltpu.VMEM((_SLICE,), jnp.float32),
            pltpu.VMEM((_L,), jnp.float32),
            pltpu.VMEM((_L,), jnp.float32),
            pltpu.VMEM((_L,), jnp.int32),
            pltpu.VMEM((_NW * _L,), jnp.float32),
            pltpu.VMEM((_NW * _L,), jnp.int32),
            pltpu.VMEM((_TAIL,), jnp.float32),
            pltpu.VMEM((_NPAIR * _L,), jnp.float32),
            pltpu.VMEM((_NPAIR * _L,), jnp.int32),
            pltpu.VMEM_SHARED((_NW * _L,), jnp.float32),
            pltpu.VMEM_SHARED((_NW * _L,), jnp.int32),
            pltpu.VMEM_SHARED((_NW * _L,), jnp.float32),
            pltpu.SemaphoreType.DMA,
            pltpu.SemaphoreType.DMA,
            pltpu.SemaphoreType.DMA,
        ],
    )


@jax.jit
def kernel(current_neuron_state, mechanism_state, prev_state):
    return _sc_kernel()(current_neuron_state, prev_state,
                        mechanism_state.reshape(-1))


# speedup vs baseline: 1.1573x; 1.0021x over previous
import dataclasses
import functools

import jax
import jax.numpy as jnp
from jax import lax
from jax.experimental import pallas as pl
from jax.experimental.pallas import tpu as pltpu
from jax.experimental.pallas import tpu_sc as plsc

_N = 4096
_M = 8
_TOPK = 10
_L = 16
_NW = 16
_SLICE = _N // _NW
_KV = _SLICE // _L
_NEG = -1.0
_BIGI = 1 << 30
_NH = 8
_NPAIR = _NH - 1
_TAIL = 3 * _TOPK + _M
_OUT = 3 * _N + _TAIL


def _sc_body(x_hbm, prev_hbm, m_hbm, out_hbm,
             xv, pv, mv, stage, nr, tvr, tir, vb, ib, tail, pr, kr,
             sh_vals, sh_idx, sh_niche, sem, semm, semo):
    cid = lax.axis_index("c")
    sid = lax.axis_index("s")
    lanes = lax.iota(jnp.int32, _L)
    lanes8 = lanes * _M

    @pl.when(cid == 0)
    def _():
        w = sid
        base = w * _SLICE
        cm = pltpu.async_copy(m_hbm.at[pl.ds(base * _M, _SLICE * _M)], mv,
                              semm)
        cx = pltpu.async_copy(x_hbm.at[pl.ds(base, _SLICE)], xv, sem)
        cp_ = pltpu.async_copy(prev_hbm.at[pl.ds(base, _SLICE)], pv, sem)
        cx.wait()
        cp_.wait()

        def scan_step(k, carry):
            tvals, tidx = carry
            off = k * _L
            ck = xv[pl.ds(off, _L)] - pv[pl.ds(off, _L)]
            stage[pl.ds(off, _L)] = ck
            sv, si = plsc.sort_key_val(jnp.abs(ck), lanes + (base + off),
                                       descending=True)
            take = sv > tvals
            nv = jnp.where(take, sv, tvals)
            ni = jnp.where(take, si, tidx)
            nv2, ni2 = plsc.sort_key_val(nv, ni)
            return (nv2, ni2)

        tvals, tidx = lax.fori_loop(
            0, _KV, scan_step,
            (jnp.full((_L,), _NEG, jnp.float32), jnp.zeros((_L,), jnp.int32)))

        co = pltpu.async_copy(stage, out_hbm.at[pl.ds(base, _SLICE)], semo)
        tvr[...] = tvals
        tir[...] = tidx
        cm.wait()

        lanehalf = lanes >> 3

        def niche_step(k, acc):
            for t in range(4):
                q = k * 4 + t
                mvv = mv[pl.ds(q * _L, _L)]
                cb = plsc.load_gather(stage, [lanehalf + 2 * q])
                acc = acc + mvv * cb
            return acc

        acc = lax.fori_loop(0, _SLICE * _M // _L // 4, niche_step,
                            jnp.zeros((_L,), jnp.float32))
        nr[...] = acc
        fold = acc + plsc.load_gather(nr, [(lanes & 7) | 8])
        part = jnp.where(lanes < _M, fold, 0.0)
        nr[...] = part
        co.wait()

        pltpu.sync_copy(nr, sh_niche.at[pl.ds(w * _L, _L)])
        pltpu.sync_copy(tvr, sh_vals.at[pl.ds(w * _L, _L)])
        pltpu.sync_copy(tir, sh_idx.at[pl.ds(w * _L, _L)])
        plsc.subcore_barrier()

        pltpu.sync_copy(sh_niche, vb)

        def nred_step(b, acc):
            return acc + vb[pl.ds(b * _L, _L)]

        niche = lax.fori_loop(0, _NW, nred_step,
                              jnp.zeros((_L,), jnp.float32))
        nr[...] = niche

        njs = [plsc.load_gather(nr, [jnp.full((_L,), j, jnp.int32)])
               for j in range(_M)]

        @pl.loop(0, _KV)
        def _(k):
            off = k * _L
            ck = stage[pl.ds(off, _L)]
            moff = lanes8 + off * _M
            ek = njs[0] * plsc.load_gather(mv, [moff])
            for j in range(1, _M):
                ek = ek + njs[j] * plsc.load_gather(mv, [moff + j])
            stage[pl.ds(off, _L)] = ek
            xv[pl.ds(off, _L)] = ck - ek

        ce = pltpu.async_copy(stage, out_hbm.at[pl.ds(_N + base, _SLICE)],
                              semo)
        cr = pltpu.async_copy(xv, out_hbm.at[pl.ds(2 * _N + base, _SLICE)],
                              semo)

        @pl.when(w == 0)
        def _():
            pltpu.sync_copy(sh_vals, vb)
            pltpu.sync_copy(sh_idx, ib)

            def merge_step(b, carry):
                tv, ti = carry
                sv, si = plsc.sort_key_val(vb[pl.ds(b * _L, _L)],
                                           ib[pl.ds(b * _L, _L)],
                                           descending=True)
                take = sv > tv
                nv = jnp.where(take, sv, tv)
                ni = jnp.where(take, si, ti)
                nv2, ni2 = plsc.sort_key_val(nv, ni)
                return (nv2, ni2)

            tv, ti = lax.fori_loop(
                0, _NW, merge_step,
                (jnp.full((_L,), _NEG, jnp.float32),
                 jnp.zeros((_L,), jnp.int32)))
            tvr[...] = tv
            tir[...] = ti

            keybase = ti * _N
            hi = lanes >= _NH

            @pl.loop(1, _NH)
            def _(s):
                perm = ((lanes + s) & (_NH - 1)) | _NH
                rv = plsc.load_gather(tvr, [perm])
                ri = plsc.load_gather(tir, [perm])
                off = (s - 1) * _L
                pr[pl.ds(off, _L)] = jnp.where(hi, tv * rv, _NEG)
                kr[pl.ds(off, _L)] = keybase + ri

            def top_step(t, carry):
                selv, selk, maxv = carry

                def max_step(s, mx):
                    return jnp.maximum(mx, pr[pl.ds(s * _L, _L)])

                m = lax.fori_loop(0, _NPAIR, max_step,
                                  jnp.full((_L,), _NEG, jnp.float32))
                ms = jnp.max(m)
                msv = jnp.full((_L,), ms)

                def key_step(s, kc):
                    off = s * _L
                    return jnp.minimum(
                        kc, jnp.where(pr[pl.ds(off, _L)] == msv,
                                      kr[pl.ds(off, _L)], _BIGI))

                kc = lax.fori_loop(0, _NPAIR, key_step,
                                   jnp.full((_L,), _BIGI, jnp.int32))
                km = jnp.min(kc)
                kmv = jnp.full((_L,), km)

                @pl.loop(0, _NPAIR)
                def _(s):
                    off = s * _L
                    pr[pl.ds(off, _L)] = jnp.where(
                        kr[pl.ds(off, _L)] == kmv, _NEG, pr[pl.ds(off, _L)])

                selv = jnp.where(lanes == t, msv, selv)
                selk = jnp.where(lanes == t, kmv, selk)
                maxv = jnp.where(t == 0, ms, maxv)
                return (selv, selk, maxv)

            selv, selk, maxv = lax.fori_loop(
                0, _TOPK, top_step,
                (jnp.zeros((_L,), jnp.float32), jnp.zeros((_L,), jnp.int32),
                 jnp.float32(0.0)))

            inv_max = jnp.full((_L,), jnp.maximum(maxv, 1e-8))
            tmask = lanes < _TOPK
            nmask = lanes < _M
            ti3 = jnp.where(tmask, lanes * 3, 0)
            tin = jnp.where(nmask, lanes + 3 * _TOPK, 0)
            plsc.store_scatter(tail, [ti3 + 2], selv / inv_max, mask=tmask)
            plsc.store_scatter(tail, [ti3],
                               (selk >> 12).astype(jnp.float32), mask=tmask)
            plsc.store_scatter(tail, [ti3 + 1],
                               (selk & (_N - 1)).astype(jnp.float32),
                               mask=tmask)
            plsc.store_scatter(tail, [tin], nr[...], mask=nmask)
            pltpu.sync_copy(tail, out_hbm.at[pl.ds(3 * _N, _TAIL)])

        ce.wait()
        cr.wait()


@functools.cache
def _sc_kernel():
    mesh = plsc.VectorSubcoreMesh(core_axis_name="c", subcore_axis_name="s")
    cp = pltpu.CompilerParams()
    if "needs_layout_passes" in pltpu.CompilerParams.__dataclass_fields__:
        cp = dataclasses.replace(cp, needs_layout_passes=False)
    return pl.kernel(
        _sc_body,
        mesh=mesh,
        compiler_params=cp,
        out_type=jax.ShapeDtypeStruct((_OUT,), jnp.float32),
        scratch_types=[
            pltpu.VMEM((_SLICE,), jnp.float32),
            pltpu.VMEM((_SLICE,), jnp.float32),
            pltpu.VMEM((_SLICE * _M,), jnp.float32),
            pltpu.VMEM((_SLICE,), jnp.float32),
            pltpu.VMEM((_L,), jnp.float32),
            pltpu.VMEM((_L,), jnp.float32),
            pltpu.VMEM((_L,), jnp.int32),
            pltpu.VMEM((_NW * _L,), jnp.float32),
            pltpu.VMEM((_NW * _L,), jnp.int32),
            pltpu.VMEM((_TAIL,), jnp.float32),
            pltpu.VMEM((_NPAIR * _L,), jnp.float32),
            pltpu.VMEM((_NPAIR * _L,), jnp.int32),
            pltpu.VMEM_SHARED((_NW * _L,), jnp.float32),
            pltpu.VMEM_SHARED((_NW * _L,), jnp.int32),
            pltpu.VMEM_SHARED((_NW * _L,), jnp.float32),
            pltpu.SemaphoreType.DMA,
            pltpu.SemaphoreType.DMA,
            pltpu.SemaphoreType.DMA,
        ],
    )


@jax.jit
def kernel(current_neuron_state, mechanism_state, prev_state):
    return _sc_kernel()(current_neuron_state, prev_state,
                        mechanism_state.reshape(-1))
